# baseline (speedup 1.0000x reference)
import functools

import jax
import jax.numpy as jnp
import numpy as np
from jax import lax
from jax.experimental import pallas as pl
from jax.experimental.pallas import tpu as pltpu
from jax.experimental.pallas import tpu_sc as plsc

H = 512
W = 512
L = 16
T_PER = 16384
NEAR = 0.1
FAR = 100.0
OCC_TH = 0.5
PRIME1 = np.uint32(2654435761).astype(np.int32)
PRIME2 = np.int32(805459861)
RES = [int(np.floor(16.0 * (1.3819 ** l))) for l in range(L)]
BMIN = (-1.5, -1.5, 0.0)
BEXT = (3.0, 3.0, 8.0)

P = 65536
NCORES = 2
NSUB = 16
NTILES = NCORES * NSUB
CHUNK = P // NTILES
NV = CHUNK // 16
CONF_CH = 512
HW = H * W
HALF = HW // NCORES
K2_CHUNK = P // NSUB
K2_ROUNDS = 8
K2_SUB = K2_CHUNK // K2_ROUNDS
OWN = HALF // NSUB
NPX3 = 256
TRASH_W = HALF * 12
ACC_W = HALF * 12 + 1024
ZERO_W = 1024

PV = 0
PP = 16
PW = 32
PB = 288
PR = 296
NPAR = 512

_f32 = jnp.float32
_i32 = jnp.int32


def _sigmoid(x):
    return 1.0 / (1.0 + jnp.exp(-x))


def _splat(ref, j):
    return plsc.load_gather(ref, [jnp.full((16,), j, _i32)])


def _bf16r(v):
    u = plsc.bitcast(v, _i32)
    r = (u + 0x7FFF + ((u >> 16) & 1)) & (-65536)
    return plsc.bitcast(r, _f32)


def _k0_body(conf_ref, og_ref):
    occ = jnp.mean(conf_ref[...], axis=1)
    og_ref[...] = occ * (1.0 / (1.0 + jnp.exp(-(occ - OCC_TH) * 10.0)))


_k0 = pl.pallas_call(
    _k0_body,
    grid=(NTILES,),
    in_specs=[pl.BlockSpec((CHUNK, 27), lambda i: (i, 0))],
    out_specs=pl.BlockSpec((CHUNK,), lambda i: (i,)),
    out_shape=jax.ShapeDtypeStruct((P,), _f32),
)


def _k1_body(means_hbm, og_hbm, tab_hbm, par_hbm,
             pay_hbm, pix_hbm, ws_hbm,
             m_v, c_v, pos_v, pf_v, pix_v, out8_v, tab_v, pay_v, ws_v,
             par_v, sem):
    cid = lax.axis_index("c")
    sid = lax.axis_index("s")
    wid = cid * NSUB + sid
    base = wid * CHUNK

    pltpu.sync_copy(par_hbm, par_v)
    pltpu.sync_copy(means_hbm.at[:, pl.ds(base, CHUNK)], m_v)

    iota = lax.iota(_i32, 16)

    vm2 = [_bf16r(_splat(par_v, PV + 8 + j)) for j in range(4)]
    pm0 = [_bf16r(_splat(par_v, PP + j)) for j in range(4)]
    pm1 = [_bf16r(_splat(par_v, PP + 4 + j)) for j in range(4)]
    pm3 = [_bf16r(_splat(par_v, PP + 12 + j)) for j in range(4)]

    pltpu.sync_copy(og_hbm.at[pl.ds(base, CHUNK)], c_v)
    if True:
        def abody(v, _):
            p0 = v * 16
            x = m_v[0, pl.ds(p0, 16)]
            y = m_v[1, pl.ds(p0, 16)]
            z = m_v[2, pl.ds(p0, 16)]
            xb, yb, zb = _bf16r(x), _bf16r(y), _bf16r(z)
            depth = vm2[0] * xb + vm2[1] * yb + vm2[2] * zb + vm2[3]
            ph0 = pm0[0] * xb + pm0[1] * yb + pm0[2] * zb + pm0[3]
            ph1 = pm1[0] * xb + pm1[1] * yb + pm1[2] * zb + pm1[3]
            ph3 = pm3[0] * xb + pm3[1] * yb + pm3[2] * zb + pm3[3]
            wclip = jnp.maximum(ph3, 1e-6)
            px = ((ph0 / wclip + 1.0) * W - 1.0) * 0.5
            py = ((ph1 / wclip + 1.0) * H - 1.0) * 0.5
            valid = ((depth > NEAR) & (depth < FAR) & (px >= 0) & (px < W)
                     & (py >= 0) & (py < H))
            pxi = jnp.clip(px.astype(_i32), 0, W - 1)
            pyi = jnp.clip(py.astype(_i32), 0, H - 1)
            pix_v[pl.ds(p0, 16)] = jnp.where(valid, pyi * W + pxi, 0)
            pos_v[pl.ds(p0, 16)] = jnp.clip((x - BMIN[0]) / BEXT[0],
                                            0.0, 1.0 - 1e-6)
            pos_v[pl.ds(CHUNK + p0, 16)] = jnp.clip((y - BMIN[1]) / BEXT[1],
                                                    0.0, 1.0 - 1e-6)
            pos_v[pl.ds(2 * CHUNK + p0, 16)] = jnp.clip(
                (z - BMIN[2]) / BEXT[2], 0.0, 1.0 - 1e-6)
            og = c_v[pl.ds(p0, 16)]
            trans = jnp.exp(-depth / FAR)
            vf = jnp.where(valid, 1.0, 0.0).astype(_f32)
            pf_v[pl.ds(p0, 16)] = og * trans * vf
            return _

        lax.fori_loop(0, NV, abody, 0)

    def zbody(i, _):
        out8_v[pl.ds(i * 16, 16)] = jnp.zeros((16,), _f32)
        return _
    lax.fori_loop(0, 8 * CHUNK // 16, zbody, 0)

    def level_body(l, _):
        pltpu.sync_copy(tab_hbm.at[pl.ds(l * (2 * T_PER), 2 * T_PER)], tab_v)
        res = _splat(par_v, PR + l)
        w0 = [_splat(par_v, PW + (2 * l) * 8 + k) for k in range(8)]
        w1 = [_splat(par_v, PW + (2 * l + 1) * 8 + k) for k in range(8)]

        def vbody(v, _):
            p0 = v * 16
            fr = []
            hx = []
            for d in range(3):
                xd = pos_v[pl.ds(d * CHUNK + p0, 16)] * res
                xi = xd.astype(_i32)
                fr.append(xd - xi.astype(_f32))
                if d == 0:
                    hx.append((xi, xi + 1))
                elif d == 1:
                    m = xi * PRIME1
                    hx.append((m, m + PRIME1))
                else:
                    m = xi * PRIME2
                    hx.append((m, m + PRIME2))
            wx = (1.0 - fr[0], fr[0])
            wy = (1.0 - fr[1], fr[1])
            wz = (1.0 - fr[2], fr[2])
            f0 = jnp.zeros((16,), _f32)
            f1 = jnp.zeros((16,), _f32)
            for dx in (0, 1):
                for dy in (0, 1):
                    wxy = wx[dx] * wy[dy]
                    hxy = hx[0][dx] ^ hx[1][dy]
                    for dz in (0, 1):
                        h = (hxy ^ hx[2][dz]) & (T_PER - 1)
                        wi = h * 2
                        g0 = plsc.load_gather(tab_v, [wi])
                        g1 = plsc.load_gather(tab_v, [wi + 1])
                        wt = wxy * wz[dz]
                        f0 = f0 + wt * g0
                        f1 = f1 + wt * g1
            for k in range(8):
                o = out8_v[pl.ds(k * CHUNK + p0, 16)]
                out8_v[pl.ds(k * CHUNK + p0, 16)] = o + f0 * w0[k] + f1 * w1[k]
            return _

        lax.fori_loop(0, NV, vbody, 0)
        return _

    lax.fori_loop(0, L, level_body, 0)

    bias = [_splat(par_v, PB + k) for k in range(8)]

    def cbody(v, _):
        p0 = v * 16
        o = [out8_v[pl.ds(k * CHUNK + p0, 16)] + bias[k] for k in range(8)]
        wgt = _sigmoid(o[3]) * pf_v[pl.ds(p0, 16)]
        wb = (p0 + iota) * 12
        ch = [wgt,
              wgt * _sigmoid(o[0]), wgt * _sigmoid(o[1]), wgt * _sigmoid(o[2])]
        ch += [wgt * o[k] for k in range(8)]
        for c in range(12):
            plsc.store_scatter(pay_v, [wb + c], ch[c])
        ws_v[pl.ds(p0, 16)] = wgt
        return _

    lax.fori_loop(0, NV, cbody, 0)

    pltpu.sync_copy(pay_v, pay_hbm.at[pl.ds(base * 12, CHUNK * 12)])
    pltpu.sync_copy(pix_v, pix_hbm.at[pl.ds(base, CHUNK)])
    pltpu.sync_copy(ws_v, ws_hbm.at[pl.ds(base, CHUNK)])


def _k2_body(pay_hbm, pix_hbm,
             accraw_hbm,
             acc_s, pay_v, idx_v, pix_v, zero_v, sem):
    cid = lax.axis_index("c")
    sid = lax.axis_index("s")
    iota = lax.iota(_i32, 16)

    def zb(i, _):
        zero_v[pl.ds(i * 16, 16)] = jnp.zeros((16,), _f32)
        return _
    lax.fori_loop(0, ZERO_W // 16, zb, 0)
    per = ACC_W // NSUB
    r0 = sid * per
    nfull = per // ZERO_W
    for b in range(nfull):
        pltpu.sync_copy(zero_v, acc_s.at[pl.ds(r0 + b * ZERO_W, ZERO_W)])
    rem = per - nfull * ZERO_W
    if rem:
        pltpu.sync_copy(zero_v.at[pl.ds(0, rem)],
                        acc_s.at[pl.ds(r0 + nfull * ZERO_W, rem)])
    plsc.subcore_barrier()

    lo = cid * HALF
    for rr in range(K2_ROUNDS):
        base = sid * K2_CHUNK + rr * K2_SUB
        pltpu.sync_copy(pix_hbm.at[pl.ds(base, K2_SUB)], pix_v)
        pltpu.sync_copy(pay_hbm.at[pl.ds(base * 12, K2_SUB * 12)], pay_v)

        def ibody(v, _):
            p0 = v * 16
            rel = pix_v[pl.ds(p0, 16)] - lo
            inh = (rel >= 0) & (rel < HALF)
            wb = jnp.where(inh, rel * 12, TRASH_W)
            for c in range(12):
                w = v * 192 + iota * 12 + c
                plsc.store_scatter(idx_v, [w // 128, w % 128], wb + c)
            return _
        lax.fori_loop(0, K2_SUB // 16, ibody, 0)

        ndma = K2_SUB * 12 // 128
        ngrp = ndma // 8

        def gbody(g, _):
            for u in range(8):
                j = g * 8 + u
                pltpu.async_copy(pay_v.at[pl.ds(j * 128, 128)],
                                 acc_s.at[idx_v.at[j]], sem, add=True)
            for u in range(8):
                j = g * 8 + u
                pltpu.make_async_copy(pay_v.at[pl.ds(j * 128, 128)],
                                      acc_s.at[idx_v.at[j]], sem).wait()
            return _
        lax.fori_loop(0, ngrp, gbody, 0)
    plsc.subcore_barrier()

    pltpu.sync_copy(acc_s.at[pl.ds(sid * OWN * 12, OWN * 12)],
                    accraw_hbm.at[pl.ds(cid * HALF * 12 + sid * OWN * 12,
                                        OWN * 12)])


def _k3_body(accraw_hbm, bg_hbm, outc_hbm, outf_hbm,
             nrm_v, outc3, outf3, bg_v, sem):
    cid = lax.axis_index("c")
    sid = lax.axis_index("s")
    wid = cid * NSUB + sid
    h0 = wid * 16
    pb = wid * (HW // NTILES)
    iota = lax.iota(_i32, 16)
    pltpu.sync_copy(bg_hbm, bg_v)
    bg = [_splat(bg_v, k) for k in range(3)]
    for r in range(8):
        pltpu.sync_copy(accraw_hbm.at[pl.ds((pb + r * 1024) * 12, 1024 * 12)],
                        nrm_v)

        def nbody(v, _, r=r):
            p0 = v * 16
            rows = p0 + iota
            wb = rows * 12
            lp = r * 1024 + p0 + iota
            hh = lp // W
            ww = lp % W
            aw = plsc.load_gather(nrm_v, [wb])
            a = jnp.clip(aw, 0.0, 1.0)
            dinv = a / (aw + 1e-8)
            onea = 1.0 - a
            for k in range(3):
                ck = plsc.load_gather(nrm_v, [wb + 1 + k])
                plsc.store_scatter(outc3,
                                   [jnp.full((16,), k, _i32), hh, ww],
                                   ck * dinv + onea * bg[k])
            for k in range(8):
                fk = plsc.load_gather(nrm_v, [wb + 4 + k])
                plsc.store_scatter(outf3,
                                   [hh, jnp.full((16,), k, _i32), ww],
                                   fk * dinv)
            return _

        lax.fori_loop(0, 1024 // 16, nbody, 0)
    pltpu.sync_copy(outc3, outc_hbm.at[:, pl.ds(h0, 16), :])
    pltpu.sync_copy(outf3, outf_hbm.at[pl.ds(h0, 16), :, :])


_mesh = plsc.VectorSubcoreMesh(core_axis_name="c", subcore_axis_name="s")

_k1 = functools.partial(
    pl.kernel,
    out_type=(jax.ShapeDtypeStruct((P * 12,), _f32),
              jax.ShapeDtypeStruct((P,), _i32),
              jax.ShapeDtypeStruct((P,), _f32)),
    mesh=_mesh,
    compiler_params=pltpu.CompilerParams(needs_layout_passes=False),
    scratch_types=[
        pltpu.VMEM((3, CHUNK), _f32),
        pltpu.VMEM((CHUNK,), _f32),
        pltpu.VMEM((3 * CHUNK,), _f32),
        pltpu.VMEM((CHUNK,), _f32),
        pltpu.VMEM((CHUNK,), _i32),
        pltpu.VMEM((8 * CHUNK,), _f32),
        pltpu.VMEM((2 * T_PER,), _f32),
        pltpu.VMEM((12 * CHUNK,), _f32),
        pltpu.VMEM((CHUNK,), _f32),
        pltpu.VMEM((NPAR,), _f32),
        pltpu.SemaphoreType.DMA,
    ],
)(_k1_body)

_k2 = functools.partial(
    pl.kernel,
    out_type=jax.ShapeDtypeStruct((HW * 12,), _f32),
    mesh=_mesh,
    compiler_params=pltpu.CompilerParams(needs_layout_passes=False),
    scratch_types=[
        pltpu.VMEM_SHARED((ACC_W,), _f32),
        pltpu.VMEM((K2_SUB * 12,), _f32),
        pltpu.VMEM((K2_SUB * 12 // 128, 128), _i32),
        pltpu.VMEM((K2_SUB,), _i32),
        pltpu.VMEM((ZERO_W,), _f32),
        pltpu.SemaphoreType.DMA,
    ],
)(_k2_body)

_k3 = functools.partial(
    pl.kernel,
    out_type=(jax.ShapeDtypeStruct((3, H, W), _f32),
              jax.ShapeDtypeStruct((H, 8, W), _f32)),
    mesh=_mesh,
    compiler_params=pltpu.CompilerParams(needs_layout_passes=False),
    scratch_types=[
        pltpu.VMEM((1024 * 12,), _f32),
        pltpu.VMEM((3, 16, W), _f32),
        pltpu.VMEM((16, 8, W), _f32),
        pltpu.VMEM((16,), _f32),
        pltpu.SemaphoreType.DMA,
    ],
)(_k3_body)


def kernel(means3D, primitive_confidences, feature_table, linear_weights,
           linear_bias, viewmatrix, projmatrix, cam_pos, bg_color):
    del cam_pos
    means_flat = means3D.T
    occ_gate = _k0(primitive_confidences)
    tab = feature_table.reshape(-1)
    res = jnp.asarray(np.asarray(RES, np.float32))
    params = jnp.concatenate([
        viewmatrix.reshape(-1), projmatrix.reshape(-1),
        linear_weights.reshape(-1), linear_bias.reshape(-1),
        res, jnp.zeros((NPAR - PR - L,), _f32),
    ]).astype(_f32)
    bg16 = jnp.concatenate([bg_color.astype(_f32), jnp.zeros((13,), _f32)])
    pay, pix, ws = _k1(means_flat, occ_gate, tab, params)
    accraw = _k2(pay, pix)
    outc, outf = _k3(accraw, bg16)
    return outc.transpose(1, 2, 0), outf.transpose(0, 2, 1), ws


# speedup vs baseline: 1.1148x; 1.1148x over previous
import functools

import jax
import jax.numpy as jnp
import numpy as np
from jax import lax
from jax.experimental import pallas as pl
from jax.experimental.pallas import tpu as pltpu
from jax.experimental.pallas import tpu_sc as plsc

H = 512
W = 512
L = 16
T_PER = 16384
NEAR = 0.1
FAR = 100.0
OCC_TH = 0.5
PRIME1 = np.uint32(2654435761).astype(np.int32)
PRIME2 = np.int32(805459861)
RES = [int(np.floor(16.0 * (1.3819 ** l))) for l in range(L)]
BMIN = (-1.5, -1.5, 0.0)
BEXT = (3.0, 3.0, 8.0)

P = 65536
NCORES = 2
NSUB = 16
NTILES = NCORES * NSUB
CHUNK = P // NTILES
NV = CHUNK // 16
CONF_CH = 512
HW = H * W
HALF = HW // NCORES
K2_CHUNK = P // NSUB
K2_ROUNDS = 8
K2_SUB = K2_CHUNK // K2_ROUNDS
OWN = HALF // NSUB
NPX3 = 256
TRASH_W = HALF * 12
ACC_W = HALF * 12 + 1024
ZERO_W = 1024

PV = 0
PP = 16
PW = 32
PB = 288
PR = 296
NPAR = 512

_f32 = jnp.float32
_i32 = jnp.int32


def _sigmoid(x):
    return 1.0 / (1.0 + jnp.exp(-x))


def _splat(ref, j):
    return plsc.load_gather(ref, [jnp.full((16,), j, _i32)])


def _bf16r(v):
    u = plsc.bitcast(v, _i32)
    r = (u + 0x7FFF + ((u >> 16) & 1)) & (-65536)
    return plsc.bitcast(r, _f32)


def _k0_body(conf_ref, og_ref):
    occ = jnp.mean(conf_ref[...], axis=0)
    og_ref[...] = occ * (1.0 / (1.0 + jnp.exp(-(occ - OCC_TH) * 10.0)))


_k0 = pl.pallas_call(
    _k0_body,
    grid=(8,),
    in_specs=[pl.BlockSpec((27, P // 8), lambda i: (0, i))],
    out_specs=pl.BlockSpec((P // 8,), lambda i: (i,)),
    out_shape=jax.ShapeDtypeStruct((P,), _f32),
)


def _k1_body(means_hbm, og_hbm, tab_hbm, par_hbm,
             pay_hbm, pix_hbm, ws_hbm,
             m_v, c_v, pos_v, pf_v, pix_v, out8_v, tab_v, pay_v, ws_v,
             par_v, sem):
    cid = lax.axis_index("c")
    sid = lax.axis_index("s")
    wid = cid * NSUB + sid
    base = wid * CHUNK

    pltpu.sync_copy(par_hbm, par_v)
    pltpu.sync_copy(means_hbm.at[:, pl.ds(base, CHUNK)], m_v)

    iota = lax.iota(_i32, 16)

    vm2 = [_bf16r(_splat(par_v, PV + 8 + j)) for j in range(4)]
    pm0 = [_bf16r(_splat(par_v, PP + j)) for j in range(4)]
    pm1 = [_bf16r(_splat(par_v, PP + 4 + j)) for j in range(4)]
    pm3 = [_bf16r(_splat(par_v, PP + 12 + j)) for j in range(4)]

    pltpu.sync_copy(og_hbm.at[pl.ds(base, CHUNK)], c_v)
    if True:
        def abody(v, _):
            p0 = v * 16
            x = m_v[0, pl.ds(p0, 16)]
            y = m_v[1, pl.ds(p0, 16)]
            z = m_v[2, pl.ds(p0, 16)]
            xb, yb, zb = _bf16r(x), _bf16r(y), _bf16r(z)
            depth = vm2[0] * xb + vm2[1] * yb + vm2[2] * zb + vm2[3]
            ph0 = pm0[0] * xb + pm0[1] * yb + pm0[2] * zb + pm0[3]
            ph1 = pm1[0] * xb + pm1[1] * yb + pm1[2] * zb + pm1[3]
            ph3 = pm3[0] * xb + pm3[1] * yb + pm3[2] * zb + pm3[3]
            wclip = jnp.maximum(ph3, 1e-6)
            px = ((ph0 / wclip + 1.0) * W - 1.0) * 0.5
            py = ((ph1 / wclip + 1.0) * H - 1.0) * 0.5
            valid = ((depth > NEAR) & (depth < FAR) & (px >= 0) & (px < W)
                     & (py >= 0) & (py < H))
            pxi = jnp.clip(px.astype(_i32), 0, W - 1)
            pyi = jnp.clip(py.astype(_i32), 0, H - 1)
            pix_v[pl.ds(p0, 16)] = jnp.where(valid, pyi * W + pxi, 0)
            pos_v[pl.ds(p0, 16)] = jnp.clip((x - BMIN[0]) / BEXT[0],
                                            0.0, 1.0 - 1e-6)
            pos_v[pl.ds(CHUNK + p0, 16)] = jnp.clip((y - BMIN[1]) / BEXT[1],
                                                    0.0, 1.0 - 1e-6)
            pos_v[pl.ds(2 * CHUNK + p0, 16)] = jnp.clip(
                (z - BMIN[2]) / BEXT[2], 0.0, 1.0 - 1e-6)
            og = c_v[pl.ds(p0, 16)]
            trans = jnp.exp(-depth / FAR)
            vf = jnp.where(valid, 1.0, 0.0).astype(_f32)
            pf_v[pl.ds(p0, 16)] = og * trans * vf
            return _

        lax.fori_loop(0, NV, abody, 0)

    def zbody(i, _):
        out8_v[pl.ds(i * 16, 16)] = jnp.zeros((16,), _f32)
        return _
    lax.fori_loop(0, 8 * CHUNK // 16, zbody, 0)

    def level_body(l, _):
        pltpu.sync_copy(tab_hbm.at[pl.ds(l * (2 * T_PER), 2 * T_PER)], tab_v)
        res = _splat(par_v, PR + l)
        w0 = [_splat(par_v, PW + (2 * l) * 8 + k) for k in range(8)]
        w1 = [_splat(par_v, PW + (2 * l + 1) * 8 + k) for k in range(8)]

        def vbody(v, _):
            p0 = v * 16
            fr = []
            hx = []
            for d in range(3):
                xd = pos_v[pl.ds(d * CHUNK + p0, 16)] * res
                xi = xd.astype(_i32)
                fr.append(xd - xi.astype(_f32))
                if d == 0:
                    hx.append((xi, xi + 1))
                elif d == 1:
                    m = xi * PRIME1
                    hx.append((m, m + PRIME1))
                else:
                    m = xi * PRIME2
                    hx.append((m, m + PRIME2))
            wx = (1.0 - fr[0], fr[0])
            wy = (1.0 - fr[1], fr[1])
            wz = (1.0 - fr[2], fr[2])
            f0 = jnp.zeros((16,), _f32)
            f1 = jnp.zeros((16,), _f32)
            for dx in (0, 1):
                for dy in (0, 1):
                    wxy = wx[dx] * wy[dy]
                    hxy = hx[0][dx] ^ hx[1][dy]
                    for dz in (0, 1):
                        h = (hxy ^ hx[2][dz]) & (T_PER - 1)
                        wi = h * 2
                        g0 = plsc.load_gather(tab_v, [wi])
                        g1 = plsc.load_gather(tab_v, [wi + 1])
                        wt = wxy * wz[dz]
                        f0 = f0 + wt * g0
                        f1 = f1 + wt * g1
            for k in range(8):
                o = out8_v[pl.ds(k * CHUNK + p0, 16)]
                out8_v[pl.ds(k * CHUNK + p0, 16)] = o + f0 * w0[k] + f1 * w1[k]
            return _

        lax.fori_loop(0, NV, vbody, 0)
        return _

    lax.fori_loop(0, L, level_body, 0)

    bias = [_splat(par_v, PB + k) for k in range(8)]

    def cbody(v, _):
        p0 = v * 16
        o = [out8_v[pl.ds(k * CHUNK + p0, 16)] + bias[k] for k in range(8)]
        wgt = _sigmoid(o[3]) * pf_v[pl.ds(p0, 16)]
        wb = (p0 + iota) * 12
        ch = [wgt,
              wgt * _sigmoid(o[0]), wgt * _sigmoid(o[1]), wgt * _sigmoid(o[2])]
        ch += [wgt * o[k] for k in range(8)]
        for c in range(12):
            plsc.store_scatter(pay_v, [wb + c], ch[c])
        ws_v[pl.ds(p0, 16)] = wgt
        return _

    lax.fori_loop(0, NV, cbody, 0)

    pltpu.sync_copy(pay_v, pay_hbm.at[pl.ds(base * 12, CHUNK * 12)])
    pltpu.sync_copy(pix_v, pix_hbm.at[pl.ds(base, CHUNK)])
    pltpu.sync_copy(ws_v, ws_hbm.at[pl.ds(base, CHUNK)])


def _k2_body(pay_hbm, pix_hbm,
             accraw_hbm,
             acc_s, pay_v, idx_v, pix_v, zero_v, sem):
    cid = lax.axis_index("c")
    sid = lax.axis_index("s")
    iota = lax.iota(_i32, 16)

    def zb(i, _):
        zero_v[pl.ds(i * 16, 16)] = jnp.zeros((16,), _f32)
        return _
    lax.fori_loop(0, ZERO_W // 16, zb, 0)
    per = ACC_W // NSUB
    r0 = sid * per
    nfull = per // ZERO_W
    for b in range(nfull):
        pltpu.sync_copy(zero_v, acc_s.at[pl.ds(r0 + b * ZERO_W, ZERO_W)])
    rem = per - nfull * ZERO_W
    if rem:
        pltpu.sync_copy(zero_v.at[pl.ds(0, rem)],
                        acc_s.at[pl.ds(r0 + nfull * ZERO_W, rem)])
    plsc.subcore_barrier()

    lo = cid * HALF
    for rr in range(K2_ROUNDS):
        base = sid * K2_CHUNK + rr * K2_SUB
        pltpu.sync_copy(pix_hbm.at[pl.ds(base, K2_SUB)], pix_v)
        pltpu.sync_copy(pay_hbm.at[pl.ds(base * 12, K2_SUB * 12)], pay_v)

        def ibody(v, _):
            p0 = v * 16
            rel = pix_v[pl.ds(p0, 16)] - lo
            inh = (rel >= 0) & (rel < HALF)
            wb = jnp.where(inh, rel * 12, TRASH_W)
            for c in range(12):
                w = v * 192 + iota * 12 + c
                plsc.store_scatter(idx_v, [w // 128, w % 128], wb + c)
            return _
        lax.fori_loop(0, K2_SUB // 16, ibody, 0)

        ndma = K2_SUB * 12 // 128
        ngrp = ndma // 8

        def gbody(g, _):
            for u in range(8):
                j = g * 8 + u
                pltpu.async_copy(pay_v.at[pl.ds(j * 128, 128)],
                                 acc_s.at[idx_v.at[j]], sem, add=True)
            for u in range(8):
                j = g * 8 + u
                pltpu.make_async_copy(pay_v.at[pl.ds(j * 128, 128)],
                                      acc_s.at[idx_v.at[j]], sem).wait()
            return _
        lax.fori_loop(0, ngrp, gbody, 0)
    plsc.subcore_barrier()

    pltpu.sync_copy(acc_s.at[pl.ds(sid * OWN * 12, OWN * 12)],
                    accraw_hbm.at[pl.ds(cid * HALF * 12 + sid * OWN * 12,
                                        OWN * 12)])


def _k3_body(accraw_hbm, bg_hbm, outc_hbm, outf_hbm,
             nrm_v, outc3, outf3, bg_v, sem):
    cid = lax.axis_index("c")
    sid = lax.axis_index("s")
    wid = cid * NSUB + sid
    h0 = wid * 16
    pb = wid * (HW // NTILES)
    iota = lax.iota(_i32, 16)
    pltpu.sync_copy(bg_hbm, bg_v)
    bg = [_splat(bg_v, k) for k in range(3)]
    for r in range(8):
        pltpu.sync_copy(accraw_hbm.at[pl.ds((pb + r * 1024) * 12, 1024 * 12)],
                        nrm_v)

        def nbody(v, _, r=r):
            p0 = v * 16
            rows = p0 + iota
            wb = rows * 12
            lp = r * 1024 + p0 + iota
            hh = lp // W
            ww = lp % W
            aw = plsc.load_gather(nrm_v, [wb])
            a = jnp.clip(aw, 0.0, 1.0)
            dinv = a / (aw + 1e-8)
            onea = 1.0 - a
            for k in range(3):
                ck = plsc.load_gather(nrm_v, [wb + 1 + k])
                plsc.store_scatter(outc3,
                                   [jnp.full((16,), k, _i32), hh, ww],
                                   ck * dinv + onea * bg[k])
            for k in range(8):
                fk = plsc.load_gather(nrm_v, [wb + 4 + k])
                plsc.store_scatter(outf3,
                                   [hh, jnp.full((16,), k, _i32), ww],
                                   fk * dinv)
            return _

        lax.fori_loop(0, 1024 // 16, nbody, 0)
    pltpu.sync_copy(outc3, outc_hbm.at[:, pl.ds(h0, 16), :])
    pltpu.sync_copy(outf3, outf_hbm.at[pl.ds(h0, 16), :, :])


_mesh = plsc.VectorSubcoreMesh(core_axis_name="c", subcore_axis_name="s")

_k1 = functools.partial(
    pl.kernel,
    out_type=(jax.ShapeDtypeStruct((P * 12,), _f32),
              jax.ShapeDtypeStruct((P,), _i32),
              jax.ShapeDtypeStruct((P,), _f32)),
    mesh=_mesh,
    compiler_params=pltpu.CompilerParams(needs_layout_passes=False),
    scratch_types=[
        pltpu.VMEM((3, CHUNK), _f32),
        pltpu.VMEM((CHUNK,), _f32),
        pltpu.VMEM((3 * CHUNK,), _f32),
        pltpu.VMEM((CHUNK,), _f32),
        pltpu.VMEM((CHUNK,), _i32),
        pltpu.VMEM((8 * CHUNK,), _f32),
        pltpu.VMEM((2 * T_PER,), _f32),
        pltpu.VMEM((12 * CHUNK,), _f32),
        pltpu.VMEM((CHUNK,), _f32),
        pltpu.VMEM((NPAR,), _f32),
        pltpu.SemaphoreType.DMA,
    ],
)(_k1_body)

_k2 = functools.partial(
    pl.kernel,
    out_type=jax.ShapeDtypeStruct((HW * 12,), _f32),
    mesh=_mesh,
    compiler_params=pltpu.CompilerParams(needs_layout_passes=False),
    scratch_types=[
        pltpu.VMEM_SHARED((ACC_W,), _f32),
        pltpu.VMEM((K2_SUB * 12,), _f32),
        pltpu.VMEM((K2_SUB * 12 // 128, 128), _i32),
        pltpu.VMEM((K2_SUB,), _i32),
        pltpu.VMEM((ZERO_W,), _f32),
        pltpu.SemaphoreType.DMA,
    ],
)(_k2_body)

_k3 = functools.partial(
    pl.kernel,
    out_type=(jax.ShapeDtypeStruct((3, H, W), _f32),
              jax.ShapeDtypeStruct((H, 8, W), _f32)),
    mesh=_mesh,
    compiler_params=pltpu.CompilerParams(needs_layout_passes=False),
    scratch_types=[
        pltpu.VMEM((1024 * 12,), _f32),
        pltpu.VMEM((3, 16, W), _f32),
        pltpu.VMEM((16, 8, W), _f32),
        pltpu.VMEM((16,), _f32),
        pltpu.SemaphoreType.DMA,
    ],
)(_k3_body)


def kernel(means3D, primitive_confidences, feature_table, linear_weights,
           linear_bias, viewmatrix, projmatrix, cam_pos, bg_color):
    del cam_pos
    means_flat = means3D.T
    occ_gate = _k0(primitive_confidences.T)
    tab = feature_table.reshape(-1)
    res = jnp.asarray(np.asarray(RES, np.float32))
    params = jnp.concatenate([
        viewmatrix.reshape(-1), projmatrix.reshape(-1),
        linear_weights.reshape(-1), linear_bias.reshape(-1),
        res, jnp.zeros((NPAR - PR - L,), _f32),
    ]).astype(_f32)
    bg16 = jnp.concatenate([bg_color.astype(_f32), jnp.zeros((13,), _f32)])
    pay, pix, ws = _k1(means_flat, occ_gate, tab, params)
    accraw = _k2(pay, pix)
    outc, outf = _k3(accraw, bg16)
    return outc.transpose(1, 2, 0), outf.transpose(0, 2, 1), ws
